# baseline (speedup 1.0000x reference)
import functools

import jax
import jax.numpy as jnp
from jax import lax
from jax.experimental import pallas as pl
from jax.experimental.pallas import tpu as pltpu
from jax.experimental.pallas import tpu_sc as plsc

_GAMMA = 2
_LAMB = max(5.0, 1500.0 / (1.0 + 0.001 * 1))
_DENOM = 1.0 + _LAMB

_B = 4096
_C = 1000
_BR = 512
_NBLK = _B // _BR

_NC = 2
_NS = 16
_NW = _NC * _NS
_PER = _B // _NW


def _rowstats_body(cos_ref, tgt_ref, m0_ref, s0_ref, ct_ref):
    cosb = cos_ref[...]
    tgt = tgt_ref[...]
    col = lax.broadcasted_iota(jnp.int32, cosb.shape, 1)
    m0 = jnp.max(cosb, axis=1, keepdims=True)
    s0 = jnp.sum(jnp.exp(cosb - m0), axis=1, keepdims=True)
    ct = jnp.sum(jnp.where(col == tgt, cosb, 0.0), axis=1, keepdims=True)
    m0_ref[...] = m0
    s0_ref[...] = s0
    ct_ref[...] = ct


def _combine_body(m0_ref, s0_ref, ct_ref, ph_ref, out_ref):
    m0 = m0_ref[...]
    s0 = s0_ref[...]
    ct = ct_ref[...]
    ph = ph_ref[...]
    mt = ct + (ph - ct) / _DENOM
    m = jnp.maximum(m0, mt)
    s = s0 * jnp.exp(m0 - m) - jnp.exp(ct - m) + jnp.exp(mt - m)
    logpt = mt - m - jnp.log(s)
    pt = jnp.exp(logpt)
    omp = 1.0 - pt
    out_ref[...] = -jnp.sum(omp * omp * logpt, keepdims=True) / _B


def _rowstats_call(cos_theta, tgt_col):
    vec = jax.ShapeDtypeStruct((_B, 1), jnp.float32)
    return pl.pallas_call(
        _rowstats_body,
        grid=(_NBLK,),
        in_specs=[
            pl.BlockSpec((_BR, _C), lambda i: (i, 0)),
            pl.BlockSpec((_BR, 1), lambda i: (i, 0)),
        ],
        out_specs=[
            pl.BlockSpec((_BR, 1), lambda i: (i, 0)),
            pl.BlockSpec((_BR, 1), lambda i: (i, 0)),
            pl.BlockSpec((_BR, 1), lambda i: (i, 0)),
        ],
        out_shape=[vec, vec, vec],
    )(cos_theta, tgt_col)


def _combine_call(m0, s0, ct, ph):
    r = pl.pallas_call(
        _combine_body,
        out_shape=jax.ShapeDtypeStruct((1, 1), jnp.float32),
    )(m0.reshape(32, 128), s0.reshape(32, 128), ct.reshape(32, 128),
      ph.reshape(32, 128))
    return r[0, 0]


@functools.cache
def _build_sc_gather():
    mesh = plsc.VectorSubcoreMesh(core_axis_name="c", subcore_axis_name="s")

    @functools.partial(
        pl.kernel,
        mesh=mesh,
        out_type=jax.ShapeDtypeStruct((_B,), jnp.float32),
        scratch_types=[
            pltpu.VMEM((_PER,), jnp.int32),
            pltpu.VMEM((_PER,), jnp.int32),
            pltpu.VMEM((_PER,), jnp.float32),
            pltpu.SemaphoreType.DMA,
        ],
    )
    def _sc_gather(phi_hbm, tgt_hbm, out_hbm, tgt_v, idx_v, val_v, sem):
        wid = lax.axis_index("s") * _NC + lax.axis_index("c")
        base = wid * _PER
        pltpu.sync_copy(tgt_hbm.at[pl.ds(base, _PER)], tgt_v)
        for j in range(_PER // 16):
            lane = lax.iota(jnp.int32, 16)
            rows = base + j * 16 + lane
            idx_v[pl.ds(j * 16, 16)] = rows * _C + tgt_v[pl.ds(j * 16, 16)]
        pltpu.async_copy(phi_hbm.at[idx_v], val_v, sem).wait()
        pltpu.sync_copy(val_v, out_hbm.at[pl.ds(base, _PER)])

    return _sc_gather


def kernel(cos_theta, phi_theta, xlen, target):
    del xlen
    tgt_col = target.reshape(_B, 1)
    m0, s0, ct = _rowstats_call(cos_theta, tgt_col)
    ph = _build_sc_gather()(phi_theta.reshape(-1), target)
    return _combine_call(m0, s0, ct, ph)


# speedup vs baseline: 1.8234x; 1.8234x over previous
import jax
import jax.numpy as jnp
from jax import lax
from jax.experimental import pallas as pl

_LAMB = max(5.0, 1500.0 / 1.001)
_DENOM = 1.0 + _LAMB
_B = 4096
_C = 1000
_BR = 512
_NBLK = _B // _BR


def _fused_body(cos_ref, phi_ref, tgt_ref, out_ref):
    i = pl.program_id(0)
    cosb = cos_ref[...]
    phib = phi_ref[...]
    tgt = tgt_ref[...]
    col = lax.broadcasted_iota(jnp.int32, cosb.shape, 1)
    mask = col == tgt
    outb = cosb + jnp.where(mask, (phib - cosb) / _DENOM, 0.0)
    m = jnp.max(outb, axis=1, keepdims=True)
    s = jnp.sum(jnp.exp(outb - m), axis=1, keepdims=True)
    mt = jnp.sum(jnp.where(mask, outb, 0.0), axis=1, keepdims=True)
    logpt = mt - m - jnp.log(s)
    pt = jnp.exp(logpt)
    omp = 1.0 - pt
    partial = -jnp.sum(omp * omp * logpt, keepdims=True) / _B

    @pl.when(i == 0)
    def _():
        out_ref[...] = jnp.zeros_like(out_ref)

    out_ref[...] += partial


def kernel(cos_theta, phi_theta, xlen, target):
    del xlen
    tgt_col = target.reshape(_B, 1)
    r = pl.pallas_call(
        _fused_body,
        grid=(_NBLK,),
        in_specs=[
            pl.BlockSpec((_BR, _C), lambda i: (i, 0)),
            pl.BlockSpec((_BR, _C), lambda i: (i, 0)),
            pl.BlockSpec((_BR, 1), lambda i: (i, 0)),
        ],
        out_specs=pl.BlockSpec((1, 1), lambda i: (0, 0)),
        out_shape=jax.ShapeDtypeStruct((1, 1), jnp.float32),
    )(cos_theta, phi_theta, tgt_col)
    return r[0, 0]
